# baseline (speedup 1.0000x reference)
import jax
import jax.numpy as jnp
from jax.experimental import pallas as pl
from jax.experimental.pallas import tpu as pltpu

BEAMS = 8
TEETH = 8
NEG = -3.0e38


def _step_kernel(ti_ref, preds_ref, blls_ref, seqs_ref,
                 blls_out_ref, nt_out_ref, seqs_out_ref):
    x4 = preds_ref[0]
    bll = blls_ref[0]
    seqs = seqs_ref[0]
    C = x4.shape[2]
    V = TEETH * C

    M1 = jnp.max(x4, axis=1)
    m = jnp.max(M1, axis=1, keepdims=True)
    m3 = jnp.expand_dims(m, 1)
    s_part = jnp.sum(jnp.exp(x4 - m3), axis=1)
    s = jnp.sum(s_part, axis=1, keepdims=True)
    lse = m + jnp.log(s)

    col = jax.lax.broadcasted_iota(jnp.int32, M1.shape, 1)
    m_cur = M1
    js = []
    for _ in range(BEAMS):
        v = jnp.max(m_cur, axis=1, keepdims=True)
        j = jnp.min(jnp.where(m_cur == v, col, C), axis=1, keepdims=True)
        m_cur = jnp.where(col == j, NEG, m_cur)
        js.append(j)
    J = jnp.concatenate(js, axis=1)

    col3 = jax.lax.broadcasted_iota(jnp.int32, (BEAMS, C, BEAMS), 1)
    J3 = jnp.expand_dims(J, 1)
    H = jnp.where(col3 == J3, 1.0, 0.0)
    Y = jax.lax.dot_general(x4, H, (((2,), (1,)), ((0,), (0,))),
                            preferred_element_type=jnp.float32,
                            precision=jax.lax.Precision.HIGHEST)

    s_iota = jax.lax.broadcasted_iota(jnp.int32, (BEAMS, TEETH, BEAMS), 1)
    idx3 = s_iota * C + J3

    vals = []
    idxs = []
    y_cur = Y
    for _ in range(BEAMS):
        va = jnp.max(y_cur, axis=2, keepdims=True)
        v = jnp.max(va, axis=1, keepdims=True)
        cand = jnp.where(y_cur == v, idx3, V)
        ci = jnp.min(jnp.min(cand, axis=2, keepdims=True),
                     axis=1, keepdims=True)
        y_cur = jnp.where(idx3 == ci, NEG, y_cur)
        vals.append(v[:, :, 0])
        idxs.append(ci[:, :, 0])
    topv = jnp.concatenate(vals, axis=1)
    BC = jnp.concatenate(idxs, axis=1)

    A = jnp.log(jnp.exp(topv - lse) + 1e-8) + bll

    F = (jax.lax.broadcasted_iota(jnp.int32, (BEAMS, BEAMS), 0) * BEAMS
         + jax.lax.broadcasted_iota(jnp.int32, (BEAMS, BEAMS), 1))
    R = jnp.zeros((BEAMS, BEAMS), jnp.int32)
    for i2 in range(BEAMS):
        for k2 in range(BEAMS):
            a = A[i2, k2]
            f = i2 * BEAMS + k2
            R = R + jnp.where((a > A) | ((a == A) & (f < F)), 1, 0)

    lane8 = jax.lax.broadcasted_iota(jnp.int32, (1, BEAMS), 1)
    row8 = jax.lax.broadcasted_iota(jnp.int32, (BEAMS, 1), 0)
    seq_col = jax.lax.broadcasted_iota(jnp.int32, (1, seqs.shape[1]), 1)
    t_pos = ti_ref[0] + 1

    new_blls = jnp.zeros((1, BEAMS), jnp.float32)
    new_nt = jnp.zeros((1, BEAMS), jnp.int32)
    for k in range(BEAMS):
        sel = R == k
        e_ll = jnp.sum(jnp.where(sel, A, 0.0))
        ntk = jnp.sum(jnp.where(sel, BC, 0))
        old = jnp.sum(jnp.where(sel, row8, 0))
        new_blls = jnp.where(lane8 == k, e_ll, new_blls)
        new_nt = jnp.where(lane8 == k, ntk, new_nt)
        picked = jnp.sum(jnp.where(row8 == old, seqs, 0),
                         axis=0, keepdims=True)
        seqs_out_ref[0, k, :] = jnp.where(seq_col == t_pos, ntk, picked)[0]

    blls_out_ref[0] = new_blls
    nt_out_ref[0] = new_nt


@jax.jit
def kernel(bpreds, blls, bseqs, ti):
    BT, V = bpreds.shape
    B = BT // BEAMS
    C = V // TEETH
    SEQ = bseqs.shape[1]
    preds = bpreds.reshape(B, BEAMS, TEETH, C)
    blls3 = blls.reshape(B, BEAMS, 1)
    seqs3 = bseqs.astype(jnp.int32).reshape(B, BEAMS, SEQ)
    ti_arr = jnp.full((1,), ti, jnp.int32)

    out = pl.pallas_call(
        _step_kernel,
        grid=(B,),
        in_specs=[
            pl.BlockSpec(memory_space=pltpu.SMEM),
            pl.BlockSpec((1, BEAMS, TEETH, C), lambda b: (b, 0, 0, 0)),
            pl.BlockSpec((1, BEAMS, 1), lambda b: (b, 0, 0)),
            pl.BlockSpec((1, BEAMS, SEQ), lambda b: (b, 0, 0)),
        ],
        out_specs=[
            pl.BlockSpec((1, 1, BEAMS), lambda b: (b, 0, 0)),
            pl.BlockSpec((1, 1, BEAMS), lambda b: (b, 0, 0)),
            pl.BlockSpec((1, BEAMS, SEQ), lambda b: (b, 0, 0)),
        ],
        out_shape=[
            jax.ShapeDtypeStruct((B, 1, BEAMS), jnp.float32),
            jax.ShapeDtypeStruct((B, 1, BEAMS), jnp.int32),
            jax.ShapeDtypeStruct((B, BEAMS, SEQ), jnp.int32),
        ],
    )(ti_arr, preds, blls3, seqs3)

    new_blls = out[0].reshape(BT)
    nt = out[1].reshape(BT)
    new_bseqs = out[2].reshape(BT, SEQ)
    return (new_blls, nt, new_bseqs)


# speedup vs baseline: 2.2852x; 2.2852x over previous
import jax
import jax.numpy as jnp
from jax.experimental import pallas as pl
from jax.experimental.pallas import tpu as pltpu

BEAMS = 8
TEETH = 8
NEG = -3.0e38


def _step_kernel(ti_ref, preds_ref, blls_ref, seqs_ref,
                 blls_out_ref, nt_out_ref, seqs_out_ref):
    x4 = preds_ref[0]
    bll = blls_ref[0]
    seqs = seqs_ref[0]
    C = x4.shape[2]
    V = TEETH * C

    M1 = jnp.max(x4, axis=1)
    m = jnp.max(M1, axis=1, keepdims=True)
    m3 = jnp.expand_dims(m, 1)
    s_part = jnp.sum(jnp.exp(x4 - m3), axis=1)
    s = jnp.sum(s_part, axis=1, keepdims=True)
    lse = m + jnp.log(s)

    col = jax.lax.broadcasted_iota(jnp.int32, M1.shape, 1)
    m_cur = M1
    js = []
    for _ in range(BEAMS):
        v = jnp.max(m_cur, axis=1, keepdims=True)
        j = jnp.min(jnp.where(m_cur == v, col, C), axis=1, keepdims=True)
        m_cur = jnp.where(col == j, NEG, m_cur)
        js.append(j)
    J = jnp.concatenate(js, axis=1)

    col_t = jax.lax.broadcasted_iota(jnp.int32, (BEAMS, 1, C), 2)
    ys = []
    for k in range(BEAMS):
        jk = jnp.expand_dims(J[:, k:k + 1], 2)
        yk = jnp.sum(jnp.where(col_t == jk, x4, 0.0),
                     axis=2, keepdims=True)
        ys.append(yk)
    Y = jnp.concatenate(ys, axis=2)

    J3 = jnp.expand_dims(J, 1)
    s_iota = jax.lax.broadcasted_iota(jnp.int32, (BEAMS, TEETH, BEAMS), 1)
    idx3 = s_iota * C + J3

    vals = []
    idxs = []
    y_cur = Y
    for _ in range(BEAMS):
        va = jnp.max(y_cur, axis=2, keepdims=True)
        v = jnp.max(va, axis=1, keepdims=True)
        cand = jnp.where(y_cur == v, idx3, V)
        ci = jnp.min(jnp.min(cand, axis=2, keepdims=True),
                     axis=1, keepdims=True)
        y_cur = jnp.where(idx3 == ci, NEG, y_cur)
        vals.append(v[:, :, 0])
        idxs.append(ci[:, :, 0])
    topv = jnp.concatenate(vals, axis=1)
    BC = jnp.concatenate(idxs, axis=1)

    A = jnp.log(jnp.exp(topv - lse) + 1e-8) + bll

    F = (jax.lax.broadcasted_iota(jnp.int32, (BEAMS, BEAMS), 0) * BEAMS
         + jax.lax.broadcasted_iota(jnp.int32, (BEAMS, BEAMS), 1))
    R = jnp.zeros((BEAMS, BEAMS), jnp.int32)
    for i2 in range(BEAMS):
        for k2 in range(BEAMS):
            a = A[i2, k2]
            f = i2 * BEAMS + k2
            R = R + jnp.where((a > A) | ((a == A) & (f < F)), 1, 0)

    lane8 = jax.lax.broadcasted_iota(jnp.int32, (1, BEAMS), 1)
    row8 = jax.lax.broadcasted_iota(jnp.int32, (BEAMS, 1), 0)
    seq_col = jax.lax.broadcasted_iota(jnp.int32, (1, seqs.shape[1]), 1)
    t_pos = ti_ref[0] + 1

    new_blls = jnp.zeros((1, BEAMS), jnp.float32)
    new_nt = jnp.zeros((1, BEAMS), jnp.int32)
    for k in range(BEAMS):
        sel = R == k
        e_ll = jnp.sum(jnp.where(sel, A, 0.0))
        ntk = jnp.sum(jnp.where(sel, BC, 0))
        old = jnp.sum(jnp.where(sel, row8, 0))
        new_blls = jnp.where(lane8 == k, e_ll, new_blls)
        new_nt = jnp.where(lane8 == k, ntk, new_nt)
        picked = jnp.sum(jnp.where(row8 == old, seqs, 0),
                         axis=0, keepdims=True)
        seqs_out_ref[0, k, :] = jnp.where(seq_col == t_pos, ntk, picked)[0]

    blls_out_ref[0] = new_blls
    nt_out_ref[0] = new_nt


@jax.jit
def kernel(bpreds, blls, bseqs, ti):
    BT, V = bpreds.shape
    B = BT // BEAMS
    C = V // TEETH
    SEQ = bseqs.shape[1]
    preds = bpreds.reshape(B, BEAMS, TEETH, C)
    blls3 = blls.reshape(B, BEAMS, 1)
    seqs3 = bseqs.astype(jnp.int32).reshape(B, BEAMS, SEQ)
    ti_arr = jnp.full((1,), ti, jnp.int32)

    out = pl.pallas_call(
        _step_kernel,
        grid=(B,),
        in_specs=[
            pl.BlockSpec(memory_space=pltpu.SMEM),
            pl.BlockSpec((1, BEAMS, TEETH, C), lambda b: (b, 0, 0, 0)),
            pl.BlockSpec((1, BEAMS, 1), lambda b: (b, 0, 0)),
            pl.BlockSpec((1, BEAMS, SEQ), lambda b: (b, 0, 0)),
        ],
        out_specs=[
            pl.BlockSpec((1, 1, BEAMS), lambda b: (b, 0, 0)),
            pl.BlockSpec((1, 1, BEAMS), lambda b: (b, 0, 0)),
            pl.BlockSpec((1, BEAMS, SEQ), lambda b: (b, 0, 0)),
        ],
        out_shape=[
            jax.ShapeDtypeStruct((B, 1, BEAMS), jnp.float32),
            jax.ShapeDtypeStruct((B, 1, BEAMS), jnp.int32),
            jax.ShapeDtypeStruct((B, BEAMS, SEQ), jnp.int32),
        ],
    )(ti_arr, preds, blls3, seqs3)

    new_blls = out[0].reshape(BT)
    nt = out[1].reshape(BT)
    new_bseqs = out[2].reshape(BT, SEQ)
    return (new_blls, nt, new_bseqs)
